# baseline (speedup 1.0000x reference)
import functools

import jax
import jax.numpy as jnp
from jax import lax
from jax.experimental import pallas as pl
from jax.experimental.pallas import tpu as pltpu
from jax.experimental.pallas import tpu_sc as plsc

B, C, H, W = 32, 256, 32, 32
HW = H * W
RW = 32
NV = C // 16
NBUF = 4


def _sc_body(x_hbm, it_hbm, o_hbm, ibuf, xbuf, obuf, isem, xsem, osem,
             *, inv_cnt):
    w = lax.axis_index("s") * 2 + lax.axis_index("c")
    r0 = w * RW

    pltpu.async_copy(it_hbm.at[pl.ds(r0, RW)], ibuf, isem).wait()

    def in_copy(b, j):
        pltpu.async_copy(x_hbm.at[pl.ds(b * HW + r0, RW)], xbuf.at[j], xsem)

    def out_copy(b, j):
        pltpu.async_copy(obuf.at[j], o_hbm.at[pl.ds(b * HW + r0, RW)], osem)

    def wait_in():
        pltpu.make_async_copy(x_hbm.at[pl.ds(0, RW)], xbuf.at[0], xsem).wait()

    def wait_out():
        pltpu.make_async_copy(obuf.at[0], o_hbm.at[pl.ds(0, RW)], osem).wait()

    shuffles = [jnp.arange(16, dtype=jnp.int32) ^ s for s in (1, 2, 4, 8)]

    def compute(j):
        def per_row(r, _):
            acc = xbuf[j, r, pl.ds(0, 16)]
            for k in range(1, NV):
                acc = acc + xbuf[j, r, pl.ds(16 * k, 16)]
            for perm in shuffles:
                acc = acc + acc.at[perm].get(mode="promise_in_bounds")
            m = acc * inv_cnt
            for k in range(NV):
                xv = xbuf[j, r, pl.ds(16 * k, 16)]
                tv = ibuf[r, pl.ds(16 * k, 16)]
                bl = xv + tv * (m - xv)
                obuf[j, r, pl.ds(16 * k, 16)] = jnp.where(bl > 0, xv, 0.0)
            return 0

        lax.fori_loop(0, RW, per_row, 0)

    for j in range(NBUF):
        in_copy(j, j)

    def quad(p, _):
        for j in range(NBUF):
            b = NBUF * p + j
            wait_in()

            @pl.when(p > 0)
            def _():
                wait_out()

            compute(j)
            out_copy(b, j)

            @pl.when(p < B // NBUF - 1)
            def _():
                in_copy(b + NBUF, j)

        return 0

    lax.fori_loop(0, B // NBUF, quad, 0)
    for _ in range(NBUF):
        wait_out()


def kernel(x, inter):
    x2 = jnp.transpose(x, (0, 2, 3, 1)).reshape(B * HW, C)
    it2 = jnp.transpose(inter, (1, 2, 0)).reshape(HW, C)
    inv_cnt = 1.0 / (C + 1e-10)
    mesh = plsc.VectorSubcoreMesh(core_axis_name="c", subcore_axis_name="s")
    k = functools.partial(
        pl.kernel,
        mesh=mesh,
        out_type=jax.ShapeDtypeStruct((B * HW, C), jnp.float32),
        scratch_types=[
            pltpu.VMEM((RW, C), jnp.float32),
            pltpu.VMEM((NBUF, RW, C), jnp.float32),
            pltpu.VMEM((NBUF, RW, C), jnp.float32),
            pltpu.SemaphoreType.DMA,
            pltpu.SemaphoreType.DMA,
            pltpu.SemaphoreType.DMA,
        ],
        compiler_params=pltpu.CompilerParams(
            use_tc_tiling_on_sc=True, needs_layout_passes=False
        ),
    )(functools.partial(_sc_body, inv_cnt=inv_cnt))
    out = k(x2, it2)
    return jnp.transpose(out.reshape(B, H, W, C), (0, 3, 1, 2))
